# baseline (speedup 1.0000x reference)
import jax
import jax.numpy as jnp
from jax import lax
from jax.experimental import pallas as pl
from jax.experimental.pallas import tpu as pltpu
from jax.experimental.pallas import tpu_sc as plsc

N = 100000
D = 128
DO = 256
NE = 100
NW = 32
NS = 16
SPAN = 3128
C = 248
CHUNKS = [C] * (SPAN // C) + ([SPAN % C] if SPAN % C else [])
OFFS = [sum(CHUNKS[:i]) for i in range(len(CHUNKS))]


def _body(element_hbm, x_hbm, table_hbm, out_hbm,
          idx_v, emb_v, table_s, x_s,
          sem_g, sem_xs, sem_w0, sem_w1, sem_xw0, sem_xw1):
    wid = lax.axis_index("s") * 2 + lax.axis_index("c")
    sid = lax.axis_index("s")
    sem_w = (sem_w0, sem_w1)
    sem_xw = (sem_xw0, sem_xw1)

    @pl.when(sid == 0)
    def _():
        pltpu.sync_copy(table_hbm, table_s)

    base = jnp.minimum(wid * SPAN, N - SPAN)
    pltpu.sync_copy(element_hbm.at[pl.ds(base, SPAN)], idx_v)
    plsc.subcore_barrier()

    def emb_write(j):
        b, off, c = j % 2, OFFS[j], CHUNKS[j]
        return pltpu.make_async_copy(
            emb_v.at[b, pl.ds(0, c), :],
            out_hbm.at[pl.ds(base + off, c), pl.ds(0, D)], sem_w[b])

    def x_write(j):
        b, off, c = j % 2, OFFS[j], CHUNKS[j]
        return pltpu.make_async_copy(
            x_s.at[sid, b, pl.ds(0, c), :],
            out_hbm.at[pl.ds(base + off, c), pl.ds(D, D)], sem_xw[b])

    for j, (off, c) in enumerate(zip(OFFS, CHUNKS)):
        b = j % 2
        if j >= 2:
            emb_write(j - 2).wait()
            x_write(j - 2).wait()
        g = pltpu.make_async_copy(
            table_s.at[idx_v.at[pl.ds(off, c)]],
            emb_v.at[b, pl.ds(0, c), :], sem_g)
        g.start()
        xs = pltpu.make_async_copy(
            x_hbm.at[pl.ds(base + off, c), :],
            x_s.at[sid, b, pl.ds(0, c), :], sem_xs)
        xs.start()
        g.wait()
        emb_write(j).start()
        xs.wait()
        x_write(j).start()

    for j in (len(CHUNKS) - 2, len(CHUNKS) - 1):
        emb_write(j).wait()
        x_write(j).wait()


@jax.jit
def _sc_embed_concat(element, x, embed_table):
    mesh = plsc.VectorSubcoreMesh(core_axis_name="c", subcore_axis_name="s")
    return pl.kernel(
        _body,
        out_type=jax.ShapeDtypeStruct((N, DO), jnp.float32),
        mesh=mesh,
        scratch_types=[
            pltpu.VMEM((SPAN,), jnp.int32),
            pltpu.VMEM((2, C, D), jnp.float32),
            pltpu.VMEM_SHARED((NE, D), jnp.float32),
            pltpu.VMEM_SHARED((NS, 2, C, D), jnp.float32),
            pltpu.SemaphoreType.DMA,
            pltpu.SemaphoreType.DMA,
            pltpu.SemaphoreType.DMA,
            pltpu.SemaphoreType.DMA,
            pltpu.SemaphoreType.DMA,
            pltpu.SemaphoreType.DMA,
        ],
    )(element, x, embed_table)


def kernel(element, x, embed_table):
    return _sc_embed_concat(element.astype(jnp.int32), x, embed_table)
